# baseline (speedup 1.0000x reference)
import functools

import jax
import jax.numpy as jnp
from jax import lax
from jax.experimental import pallas as pl
from jax.experimental.pallas import tpu as pltpu
from jax.experimental.pallas import tpu_sc as plsc

N = 10000
NP = 10240
E = 320000
D = 128

NC = 2
NS = 16
NW = NC * NS
EPW = E // NW
CH = 40
NCHUNK = EPW // CH
RPT = NP // NS
ZR = 128
HW = 16

_MESH = plsc.VectorSubcoreMesh(core_axis_name="c", subcore_axis_name="s",
                               num_cores=NC, num_subcores=NS)


def _deg_body(src_hbm, dst_hbm, hout_hbm, hin_hbm, sidx, didx, ho, hi):
    c = lax.axis_index("c")
    s = lax.axis_index("s")
    wid = c * NS + s

    @pl.loop(0, NP // 16)
    def _zero(i):
        ho[pl.ds(i * 16, 16)] = jnp.zeros((16,), jnp.float32)
        hi[pl.ds(i * 16, 16)] = jnp.zeros((16,), jnp.float32)

    pltpu.sync_copy(src_hbm.at[wid], sidx)
    pltpu.sync_copy(dst_hbm.at[wid], didx)
    ones = jnp.ones((16,), jnp.float32)

    @pl.loop(0, EPW // 16)
    def _count(q):
        plsc.addupdate_scatter(ho, [sidx[pl.ds(q * 16, 16)]], ones)
        plsc.addupdate_scatter(hi, [didx[pl.ds(q * 16, 16)]], ones)

    pltpu.sync_copy(ho, hout_hbm.at[wid, :])
    pltpu.sync_copy(hi, hin_hbm.at[wid, :])


_deg_kernel = functools.partial(
    pl.kernel,
    out_type=[jax.ShapeDtypeStruct((NW, NP), jnp.float32),
              jax.ShapeDtypeStruct((NW, NP), jnp.float32)],
    mesh=_MESH,
    compiler_params=pltpu.CompilerParams(needs_layout_passes=False),
    scratch_types=[
        pltpu.VMEM((EPW,), jnp.int32),
        pltpu.VMEM((EPW,), jnp.int32),
        pltpu.VMEM((NP,), jnp.float32),
        pltpu.VMEM((NP,), jnp.float32),
    ],
)(_deg_body)


R = 4
G = 3
DS = R - G
RP = 6
NPAIR = NCHUNK // 2
UNROLL = 12


def _agg_body(x_hbm, src_hbm, dst_hbm, out_hbm,
              sidx_v, didx_v, rows, agg_sh, gsem, ssem, sisem, disem, zsem):
    c = lax.axis_index("c")
    s = lax.axis_index("s")
    wid = c * NS + s

    @pl.loop(0, CH)
    def _fill_zeros(r):
        @pl.loop(0, D // 16)
        def _inner(q):
            rows[0, r, pl.ds(q * 16, 16)] = jnp.zeros((16,), jnp.float32)

    rbase = s * RPT

    @pl.loop(0, RPT // CH)
    def _zero_acc(j):
        pltpu.async_copy(rows.at[0], agg_sh.at[pl.ds(rbase + j * CH, CH), :],
                         zsem)

    def spair_copy(q, slot):
        return pltpu.make_async_copy(src_hbm.at[wid, q], sidx_v.at[slot],
                                     sisem.at[slot])

    def dpair_copy(q, slot):
        return pltpu.make_async_copy(dst_hbm.at[wid, q], didx_v.at[slot],
                                     disem.at[slot])

    def start_pair(q, slot):
        pltpu.async_copy(src_hbm.at[wid, q], sidx_v.at[slot], sisem.at[slot])
        pltpu.async_copy(dst_hbm.at[wid, q], didx_v.at[slot], disem.at[slot])

    def wait_pair(q, slot):
        spair_copy(q, slot).wait()
        dpair_copy(q, slot).wait()

    def start_gather(pslot, pos, m):
        pltpu.async_copy(x_hbm.at[sidx_v.at[pslot, pos]], rows.at[m],
                         gsem.at[m])

    def wait_gather(pslot, pos, m):
        pltpu.make_async_copy(x_hbm.at[sidx_v.at[pslot, pos]], rows.at[m],
                              gsem.at[m]).wait()

    def start_scatter(pslot, pos, m):
        pltpu.async_copy(rows.at[m], agg_sh.at[didx_v.at[pslot, pos]],
                         ssem.at[m], add=True)

    def wait_scatter(pslot, pos, m):
        pltpu.make_async_copy(rows.at[m], agg_sh.at[didx_v.at[pslot, pos]],
                              ssem.at[m]).wait()

    def chunk_ops(k, b, has_issue=True, has_wait=True, has_next=True,
                  has_prev=True):
        m = b % R
        wait_gather((b // 2) % RP, b % 2, m)
        start_scatter((b // 2) % RP, b % 2, m)
        if has_prev:
            wait_scatter(((b - 1) // 2) % RP, (b - 1) % 2, (b - 1) % R)
        if b % 2 == 0:
            if has_issue:
                start_pair(k // 2 + 5, (b // 2 + 5) % RP)
        else:
            if has_wait:
                wait_pair((k + 3) // 2, ((b + 3) // 2) % RP)
        if has_next:
            start_gather(((b + 3) // 2) % RP, (b + 1) % 2, (b + 3) % R)

    for q in range(RP - 1):
        start_pair(q, q)

    @pl.loop(0, RPT // CH)
    def _zero_drain(j):
        pltpu.make_async_copy(rows.at[0],
                              agg_sh.at[pl.ds(rbase + j * CH, CH), :],
                              zsem).wait()

    plsc.subcore_barrier()
    for q in range(2):
        wait_pair(q, q)
    for j in range(G):
        start_gather(j // 2, j % 2, j % R)
    for k in range(UNROLL):
        chunk_ops(k, k, has_prev=(k >= DS))

    B_END = UNROLL * ((NCHUNK - UNROLL) // UNROLL)

    @pl.loop(UNROLL, B_END, step=UNROLL)
    def _steady(base):
        for b in range(UNROLL):
            chunk_ops(base + b, b)

    for k in range(B_END, NCHUNK):
        chunk_ops(k, k % UNROLL,
                  has_issue=(k + 10 < NCHUNK),
                  has_wait=(k + 3 < NCHUNK),
                  has_next=(k + G < NCHUNK))
    for k in range(NCHUNK - DS, NCHUNK):
        b = k % UNROLL
        wait_scatter((b // 2) % RP, b % 2, b % R)

    plsc.subcore_barrier()
    pltpu.sync_copy(agg_sh.at[pl.ds(rbase, RPT), :],
                    out_hbm.at[c, pl.ds(rbase, RPT), :])


_agg_kernel = functools.partial(
    pl.kernel,
    out_type=jax.ShapeDtypeStruct((NC, NP, D), jnp.float32),
    mesh=_MESH,
    scratch_types=[
        pltpu.VMEM((RP, 2, CH), jnp.int32),
        pltpu.VMEM((RP, 2, CH), jnp.int32),
        pltpu.VMEM((R, CH, D), jnp.float32),
        pltpu.VMEM_SHARED((NP, D), jnp.float32),
        pltpu.SemaphoreType.DMA((R,)),
        pltpu.SemaphoreType.DMA((R,)),
        pltpu.SemaphoreType.DMA((RP,)),
        pltpu.SemaphoreType.DMA((RP,)),
        pltpu.SemaphoreType.DMA,
    ],
)(_agg_body)


RB = 2048


def _prep_body(ho_ref, hi_ref, h_ref, nin_ref, nout_ref, x0_ref):
    deg_o = jnp.sum(ho_ref[...], axis=0, keepdims=True)
    deg_i = jnp.sum(hi_ref[...], axis=0, keepdims=True)
    no = jnp.transpose(lax.rsqrt(jnp.maximum(deg_o, 1.0)))
    ni = jnp.transpose(lax.rsqrt(jnp.maximum(deg_i, 1.0)))
    nout_ref[...] = no
    nin_ref[...] = ni
    x0_ref[...] = h_ref[...] * no


_prep_kernel = pl.pallas_call(
    _prep_body,
    grid=(NP // RB,),
    in_specs=[
        pl.BlockSpec((NW, RB), lambda i: (0, i)),
        pl.BlockSpec((NW, RB), lambda i: (0, i)),
        pl.BlockSpec((RB, D), lambda i: (i, 0)),
    ],
    out_specs=[
        pl.BlockSpec((RB, 1), lambda i: (i, 0)),
        pl.BlockSpec((RB, 1), lambda i: (i, 0)),
        pl.BlockSpec((RB, D), lambda i: (i, 0)),
    ],
    out_shape=[
        jax.ShapeDtypeStruct((NP, 1), jnp.float32),
        jax.ShapeDtypeStruct((NP, 1), jnp.float32),
        jax.ShapeDtypeStruct((NP, D), jnp.float32),
    ],
)


def _layer_body_mid(p_ref, nin_ref, nout_ref, w_ref, b_ref, xn_ref):
    xb = (p_ref[0] + p_ref[1]) * nin_ref[...]
    y = jnp.dot(xb, w_ref[...], preferred_element_type=jnp.float32) + b_ref[...]
    xn_ref[...] = y * nout_ref[...]


def _layer_body_last(p_ref, nin_ref, nout_ref, w_ref, b_ref, y_ref):
    xb = (p_ref[0] + p_ref[1]) * nin_ref[...]
    y_ref[...] = (jnp.dot(xb, w_ref[...], preferred_element_type=jnp.float32)
                  + b_ref[...])


_layer_in_specs = [
    pl.BlockSpec((NC, RB, D), lambda i: (0, i, 0)),
    pl.BlockSpec((RB, 1), lambda i: (i, 0)),
    pl.BlockSpec((RB, 1), lambda i: (i, 0)),
    pl.BlockSpec((D, D), lambda i: (0, 0)),
    pl.BlockSpec((1, D), lambda i: (0, 0)),
]

_layer_kernel_mid = pl.pallas_call(
    _layer_body_mid,
    grid=(NP // RB,),
    in_specs=_layer_in_specs,
    out_specs=pl.BlockSpec((RB, D), lambda i: (i, 0)),
    out_shape=jax.ShapeDtypeStruct((NP, D), jnp.float32),
)

_layer_kernel_last = pl.pallas_call(
    _layer_body_last,
    grid=(NP // RB,),
    in_specs=_layer_in_specs,
    out_specs=pl.BlockSpec((RB, D), lambda i: (i, 0)),
    out_shape=jax.ShapeDtypeStruct((NP, D), jnp.float32),
)


def kernel(h, edge_index, W0, b0, W1, b1, W2, b2):
    src = edge_index[0].reshape(NW, NPAIR, 2, CH)
    dst = edge_index[1].reshape(NW, NPAIR, 2, CH)
    src_flat = edge_index[0].reshape(NW, EPW)
    dst_flat = edge_index[1].reshape(NW, EPW)
    h_pad = jnp.pad(h, ((0, NP - N), (0, 0)))

    hout_p, hin_p = _deg_kernel(src_flat, dst_flat)
    nin, nout, x = _prep_kernel(hout_p, hin_p, h_pad)

    p = _agg_kernel(x, src, dst)
    x = _layer_kernel_mid(p, nin, nout, W0, b0.reshape(1, D))

    p = _agg_kernel(x, src, dst)
    x = _layer_kernel_mid(p, nin, nout, W1, b1.reshape(1, D))

    p = _agg_kernel(x, src, dst)
    y = _layer_kernel_last(p, nin, nout, W2, b2.reshape(1, D))
    return y[:N]


# speedup vs baseline: 1.0235x; 1.0235x over previous
import functools

import jax
import jax.numpy as jnp
from jax import lax
from jax.experimental import pallas as pl
from jax.experimental.pallas import tpu as pltpu
from jax.experimental.pallas import tpu_sc as plsc

N = 10000
NP = 10240
E = 320000
D = 128

NC = 2
NS = 16
NW = NC * NS
EPW = E // NW
CH = 40
NCHUNK = EPW // CH
RPT = NP // NS
ZR = 128
HW = 16

_MESH = plsc.VectorSubcoreMesh(core_axis_name="c", subcore_axis_name="s",
                               num_cores=NC, num_subcores=NS)


def _deg_body(src_hbm, dst_hbm, hout_hbm, hin_hbm, sidx, didx, ho, hi):
    c = lax.axis_index("c")
    s = lax.axis_index("s")
    wid = c * NS + s

    @pl.loop(0, NP // 16)
    def _zero(i):
        ho[pl.ds(i * 16, 16)] = jnp.zeros((16,), jnp.float32)
        hi[pl.ds(i * 16, 16)] = jnp.zeros((16,), jnp.float32)

    pltpu.sync_copy(src_hbm.at[wid], sidx)
    pltpu.sync_copy(dst_hbm.at[wid], didx)
    ones = jnp.ones((16,), jnp.float32)

    @pl.loop(0, EPW // 16)
    def _count(q):
        plsc.addupdate_scatter(ho, [sidx[pl.ds(q * 16, 16)]], ones)
        plsc.addupdate_scatter(hi, [didx[pl.ds(q * 16, 16)]], ones)

    pltpu.sync_copy(ho, hout_hbm.at[wid, :])
    pltpu.sync_copy(hi, hin_hbm.at[wid, :])


_deg_kernel = functools.partial(
    pl.kernel,
    out_type=[jax.ShapeDtypeStruct((NW, NP), jnp.float32),
              jax.ShapeDtypeStruct((NW, NP), jnp.float32)],
    mesh=_MESH,
    compiler_params=pltpu.CompilerParams(needs_layout_passes=False),
    scratch_types=[
        pltpu.VMEM((EPW,), jnp.int32),
        pltpu.VMEM((EPW,), jnp.int32),
        pltpu.VMEM((NP,), jnp.float32),
        pltpu.VMEM((NP,), jnp.float32),
    ],
)(_deg_body)


R = 4
G = 3
DS = R - G
RI = 8


def _agg_body(x_hbm, src_hbm, dst_hbm, out_hbm,
              sidx_v, didx_v, rows, agg_sh, gsem, ssem, sisem, disem, zsem):
    c = lax.axis_index("c")
    s = lax.axis_index("s")
    wid = c * NS + s
    ebase = wid * EPW

    @pl.loop(0, CH)
    def _fill_zeros(r):
        @pl.loop(0, D // 16)
        def _inner(q):
            rows[0, r, pl.ds(q * 16, 16)] = jnp.zeros((16,), jnp.float32)

    rbase = s * RPT

    @pl.loop(0, RPT // CH)
    def _zero_acc(j):
        pltpu.async_copy(rows.at[0], agg_sh.at[pl.ds(rbase + j * CH, CH), :],
                         zsem)

    def sidx_copy(j, slot):
        return pltpu.make_async_copy(src_hbm.at[pl.ds(ebase + j * CH, CH)],
                                     sidx_v.at[slot], sisem.at[slot])

    def didx_copy(j, slot):
        return pltpu.make_async_copy(dst_hbm.at[pl.ds(ebase + j * CH, CH)],
                                     didx_v.at[slot], disem.at[slot])

    def start_sidx(j, slot):
        pltpu.async_copy(src_hbm.at[pl.ds(ebase + j * CH, CH)],
                         sidx_v.at[slot], sisem.at[slot])

    def start_didx(j, slot):
        pltpu.async_copy(dst_hbm.at[pl.ds(ebase + j * CH, CH)],
                         didx_v.at[slot], disem.at[slot])

    def start_gather(islot, m):
        pltpu.async_copy(x_hbm.at[sidx_v.at[islot]], rows.at[m],
                         gsem.at[m])

    def wait_gather(islot, m):
        pltpu.make_async_copy(x_hbm.at[sidx_v.at[islot]], rows.at[m],
                              gsem.at[m]).wait()

    def start_scatter(islot, m):
        pltpu.async_copy(rows.at[m], agg_sh.at[didx_v.at[islot]],
                         ssem.at[m], add=True)

    def wait_scatter(islot, m):
        pltpu.make_async_copy(rows.at[m], agg_sh.at[didx_v.at[islot]],
                              ssem.at[m]).wait()

    def chunk_ops(k, b8, has_sidx=True, has_didx=True, has_next=True,
                  has_prev=True):
        m = b8 % R
        wait_gather(b8, m)
        if has_sidx:
            start_sidx(k + RI, b8)
        didx_copy(k, b8).wait()
        start_scatter(b8, m)
        if has_prev:
            wait_scatter((b8 - DS) % RI, (b8 - DS) % R)
        if has_didx:
            start_didx(k + RI - DS, (b8 - DS) % RI)
        if has_next:
            sidx_copy(k + G, (b8 + G) % RI).wait()
            start_gather((b8 + G) % RI, (b8 + G) % R)

    for j in range(RI):
        start_sidx(j, j)
    for j in range(RI - DS):
        start_didx(j, j)

    @pl.loop(0, RPT // CH)
    def _zero_drain(j):
        pltpu.make_async_copy(rows.at[0],
                              agg_sh.at[pl.ds(rbase + j * CH, CH), :],
                              zsem).wait()

    plsc.subcore_barrier()
    for j in range(G):
        sidx_copy(j, j).wait()
        start_gather(j, j % R)
    for k in range(RI):
        chunk_ops(k, k, has_prev=(k >= DS))

    B_END = RI * ((NCHUNK - RI) // RI)

    @pl.loop(RI, B_END, step=RI)
    def _steady(base):
        for b in range(RI):
            chunk_ops(base + b, b)

    for k in range(B_END, NCHUNK):
        chunk_ops(k, k % RI,
                  has_sidx=(k + RI < NCHUNK),
                  has_didx=(k + RI - DS < NCHUNK),
                  has_next=(k + G < NCHUNK))
    for k in range(NCHUNK - DS, NCHUNK):
        wait_scatter(k % RI, k % R)

    plsc.subcore_barrier()
    pltpu.sync_copy(agg_sh.at[pl.ds(rbase, RPT), :],
                    out_hbm.at[c, pl.ds(rbase, RPT), :])


_agg_kernel = functools.partial(
    pl.kernel,
    out_type=jax.ShapeDtypeStruct((NC, NP, D), jnp.float32),
    mesh=_MESH,
    scratch_types=[
        pltpu.VMEM((RI, CH), jnp.int32),
        pltpu.VMEM((RI, CH), jnp.int32),
        pltpu.VMEM((R, CH, D), jnp.float32),
        pltpu.VMEM_SHARED((NP, D), jnp.float32),
        pltpu.SemaphoreType.DMA((R,)),
        pltpu.SemaphoreType.DMA((R,)),
        pltpu.SemaphoreType.DMA((RI,)),
        pltpu.SemaphoreType.DMA((RI,)),
        pltpu.SemaphoreType.DMA,
    ],
)(_agg_body)


RB = 2048


def _prep_body(ho_ref, hi_ref, h_ref, nin_ref, nout_ref, x0_ref):
    deg_o = jnp.sum(ho_ref[...], axis=0, keepdims=True)
    deg_i = jnp.sum(hi_ref[...], axis=0, keepdims=True)
    no = jnp.transpose(lax.rsqrt(jnp.maximum(deg_o, 1.0)))
    ni = jnp.transpose(lax.rsqrt(jnp.maximum(deg_i, 1.0)))
    nout_ref[...] = no
    nin_ref[...] = ni
    x0_ref[...] = h_ref[...] * no


_prep_kernel = pl.pallas_call(
    _prep_body,
    grid=(NP // RB,),
    in_specs=[
        pl.BlockSpec((NW, RB), lambda i: (0, i)),
        pl.BlockSpec((NW, RB), lambda i: (0, i)),
        pl.BlockSpec((RB, D), lambda i: (i, 0)),
    ],
    out_specs=[
        pl.BlockSpec((RB, 1), lambda i: (i, 0)),
        pl.BlockSpec((RB, 1), lambda i: (i, 0)),
        pl.BlockSpec((RB, D), lambda i: (i, 0)),
    ],
    out_shape=[
        jax.ShapeDtypeStruct((NP, 1), jnp.float32),
        jax.ShapeDtypeStruct((NP, 1), jnp.float32),
        jax.ShapeDtypeStruct((NP, D), jnp.float32),
    ],
)


def _layer_body_mid(p_ref, nin_ref, nout_ref, w_ref, b_ref, xn_ref):
    xb = (p_ref[0] + p_ref[1]) * nin_ref[...]
    y = jnp.dot(xb, w_ref[...], preferred_element_type=jnp.float32) + b_ref[...]
    xn_ref[...] = y * nout_ref[...]


def _layer_body_last(p_ref, nin_ref, nout_ref, w_ref, b_ref, y_ref):
    xb = (p_ref[0] + p_ref[1]) * nin_ref[...]
    y_ref[...] = (jnp.dot(xb, w_ref[...], preferred_element_type=jnp.float32)
                  + b_ref[...])


_layer_in_specs = [
    pl.BlockSpec((NC, RB, D), lambda i: (0, i, 0)),
    pl.BlockSpec((RB, 1), lambda i: (i, 0)),
    pl.BlockSpec((RB, 1), lambda i: (i, 0)),
    pl.BlockSpec((D, D), lambda i: (0, 0)),
    pl.BlockSpec((1, D), lambda i: (0, 0)),
]

_layer_kernel_mid = pl.pallas_call(
    _layer_body_mid,
    grid=(NP // RB,),
    in_specs=_layer_in_specs,
    out_specs=pl.BlockSpec((RB, D), lambda i: (i, 0)),
    out_shape=jax.ShapeDtypeStruct((NP, D), jnp.float32),
)

_layer_kernel_last = pl.pallas_call(
    _layer_body_last,
    grid=(NP // RB,),
    in_specs=_layer_in_specs,
    out_specs=pl.BlockSpec((RB, D), lambda i: (i, 0)),
    out_shape=jax.ShapeDtypeStruct((NP, D), jnp.float32),
)


def kernel(h, edge_index, W0, b0, W1, b1, W2, b2):
    src = edge_index[0]
    dst = edge_index[1]
    src_flat = edge_index[0].reshape(NW, EPW)
    dst_flat = edge_index[1].reshape(NW, EPW)
    h_pad = jnp.pad(h, ((0, NP - N), (0, 0)))

    hout_p, hin_p = _deg_kernel(src_flat, dst_flat)
    nin, nout, x = _prep_kernel(hout_p, hin_p, h_pad)

    p = _agg_kernel(x, src, dst)
    x = _layer_kernel_mid(p, nin, nout, W0, b0.reshape(1, D))

    p = _agg_kernel(x, src, dst)
    x = _layer_kernel_mid(p, nin, nout, W1, b1.reshape(1, D))

    p = _agg_kernel(x, src, dst)
    y = _layer_kernel_last(p, nin, nout, W2, b2.reshape(1, D))
    return y[:N]
